# baseline (speedup 1.0000x reference)
import functools

import jax
import jax.numpy as jnp
from jax import lax
from jax.experimental import pallas as pl
from jax.experimental.pallas import tpu as pltpu
from jax.experimental.pallas import tpu_sc as plsc

NC = 2
NS = 16
L = 16
NW = NC * NS

G = 128
EMBED = 64
PAIR = 2 * EMBED


def _sc_gather_dot(uidx2d, midx2d, ue2, me2, ub_flat, mb_flat, batch):
  chunk = batch // NW
  nsplit = chunk // G
  mesh = plsc.VectorSubcoreMesh(core_axis_name="c", subcore_axis_name="s")

  @functools.partial(
      pl.kernel,
      out_type=(
          jax.ShapeDtypeStruct((NW, L), jnp.float32),
          jax.ShapeDtypeStruct((batch // G, G), jnp.float32)
      ),
      mesh=mesh,
      compiler_params=pltpu.CompilerParams(
          use_tc_tiling_on_sc=True, needs_layout_passes=False),
      scratch_types=[
          pltpu.VMEM((nsplit, G), jnp.int32),
          pltpu.VMEM((nsplit, G), jnp.int32),
          pltpu.VMEM((nsplit, G), jnp.int32),
          pltpu.VMEM((nsplit, G), jnp.int32),
          pltpu.VMEM((2, G, PAIR), jnp.float32),
          pltpu.VMEM((2, G, PAIR), jnp.float32),
          pltpu.VMEM((nsplit, G), jnp.float32),
          pltpu.VMEM((nsplit, G), jnp.float32),
          pltpu.VMEM((L,), jnp.float32),
          pltpu.SemaphoreType.DMA,
          pltpu.SemaphoreType.DMA,
          pltpu.SemaphoreType.DMA,
      ],
  )
  def k1(uidx_hbm, midx_hbm, ue_hbm, me_hbm, ub_hbm, mb_hbm,
         part_hbm, bsum_hbm,
         uidx_v, midx_v, uhalf_v, mhalf_v, urows_v, mrows_v, ubv, mbv, accv,
         sem_u, sem_m, sem_b):
    wid = lax.axis_index("s") * NC + lax.axis_index("c")
    row0 = wid * nsplit
    pltpu.sync_copy(uidx_hbm.at[pl.ds(row0, nsplit)], uidx_v)
    pltpu.sync_copy(midx_hbm.at[pl.ds(row0, nsplit)], midx_v)

    for j in range(nsplit):
      for s in range(G // L):
        sl = pl.ds(s * L, L)
        uhalf_v[j, sl] = lax.shift_right_logical(uidx_v[j, sl], 1)
        mhalf_v[j, sl] = lax.shift_right_logical(midx_v[j, sl], 1)

    bias_copies = []
    for j in range(nsplit):
      bias_copies.append(pltpu.async_copy(
          ub_hbm.at[uidx_v.at[j]], ubv.at[j], sem_b))
      bias_copies.append(pltpu.async_copy(
          mb_hbm.at[midx_v.at[j]], mbv.at[j], sem_b))

    def gather(j, buf):
      cu = pltpu.async_copy(ue_hbm.at[uhalf_v.at[j]], urows_v.at[buf], sem_u)
      cm = pltpu.async_copy(me_hbm.at[mhalf_v.at[j]], mrows_v.at[buf], sem_m)
      return cu, cm

    pend = gather(0, 0)
    zero = jnp.zeros((L,), jnp.float32)
    acc = (zero, zero)
    lane = lax.iota(jnp.int32, L)
    for j in range(nsplit):
      buf = j % 2
      pend[0].wait()
      pend[1].wait()
      if j + 1 < nsplit:
        pend = gather(j + 1, (j + 1) % 2)
      a0, a1 = acc
      for g in range(G // L):
        sl = pl.ds(g * L, L)
        ridx = lane + (g * L)
        ucol0 = lax.shift_left(jnp.bitwise_and(uidx_v[j, sl], 1), 6)
        mcol0 = lax.shift_left(jnp.bitwise_and(midx_v[j, sl], 1), 6)

        def kbody(k, carry):
          b0, b1, uc, mc = carry
          u0 = plsc.load_gather(urows_v.at[buf], [ridx, uc])
          m0 = plsc.load_gather(mrows_v.at[buf], [ridx, mc])
          u1 = plsc.load_gather(urows_v.at[buf], [ridx, uc + 1])
          m1 = plsc.load_gather(mrows_v.at[buf], [ridx, mc + 1])
          return (b0 + u0 * m0, b1 + u1 * m1, uc + 2, mc + 2)

        a0, a1, _, _ = lax.fori_loop(
            0, EMBED // 2, kbody, (a0, a1, ucol0, mcol0), unroll=8)
      acc = (a0, a1)

    for c in bias_copies:
      c.wait()
    for j in range(nsplit):
      for s in range(G // L):
        sl = pl.ds(s * L, L)
        ubv[j, sl] = ubv[j, sl] + mbv[j, sl]
    pltpu.sync_copy(ubv, bsum_hbm.at[pl.ds(row0, nsplit)])

    accv[...] = acc[0] + acc[1]
    pltpu.sync_copy(accv, part_hbm.at[wid])

  return k1(uidx2d, midx2d, ue2, me2, ub_flat, mb_flat)


def _tc_finish(part_ref, bsum_ref, out_ref):
  s = jnp.sum(part_ref[...])
  out_ref[...] = jax.nn.sigmoid(bsum_ref[...] + s)


def kernel(inputs, user_embedding, user_bias, movie_embedding, movie_bias):
  batch = inputs.shape[0]
  uidx2d = inputs[:, 0].reshape(batch // G, G)
  midx2d = inputs[:, 1].reshape(batch // G, G)
  ue2 = user_embedding.reshape(-1, PAIR)
  me2 = movie_embedding.reshape(-1, PAIR)
  ub_flat = user_bias.reshape(-1)
  mb_flat = movie_bias.reshape(-1)

  partials, bsum = _sc_gather_dot(
      uidx2d, midx2d, ue2, me2, ub_flat, mb_flat, batch)

  out = pl.pallas_call(
      _tc_finish,
      out_shape=jax.ShapeDtypeStruct(bsum.shape, jnp.float32),
  )(partials, bsum)
  return out.reshape(batch, 1)


# speedup vs baseline: 1.4977x; 1.4977x over previous
import functools

import jax
import jax.numpy as jnp
from jax import lax
from jax.experimental import pallas as pl
from jax.experimental.pallas import tpu as pltpu
from jax.experimental.pallas import tpu_sc as plsc

NC = 2
NS = 16
L = 16
NW = NC * NS

G = 128
EMBED = 64
PAIR = 2 * EMBED
XCOLS = 2176
NBLK = 23
SPLIT = NBLK * XCOLS


def _pack_body(ua_ref, ub_ref, ma_ref, mb_ref, pu_ref, pm_ref):
  pu_ref[:, 0:EMBED] = jnp.transpose(ua_ref[...])
  pu_ref[:, EMBED:PAIR] = jnp.transpose(ub_ref[...])
  pm_ref[:, 0:EMBED] = jnp.transpose(ma_ref[...])
  pm_ref[:, EMBED:PAIR] = jnp.transpose(mb_ref[...])


def _tc_pack(ue_t, me_t):
  lo = pl.BlockSpec((EMBED, XCOLS), lambda b: (0, b))
  hi = pl.BlockSpec((EMBED, XCOLS), lambda b: (0, b + NBLK))
  out = pl.BlockSpec((XCOLS, PAIR), lambda b: (b, 0))
  return pl.pallas_call(
      _pack_body,
      grid=(NBLK,),
      in_specs=[lo, hi, lo, hi],
      out_specs=[out, out],
      out_shape=[jax.ShapeDtypeStruct((SPLIT, PAIR), jnp.float32)] * 2,
  )(ue_t, ue_t, me_t, me_t)


def _sc_gather_dot(uidx2d, midx2d, pu, pm, ub_flat, mb_flat, batch):
  chunk = batch // NW
  nsplit = chunk // G
  mesh = plsc.VectorSubcoreMesh(core_axis_name="c", subcore_axis_name="s")

  @functools.partial(
      pl.kernel,
      out_type=(
          jax.ShapeDtypeStruct((NW, L), jnp.float32),
          jax.ShapeDtypeStruct((batch // G, G), jnp.float32)
      ),
      mesh=mesh,
      compiler_params=pltpu.CompilerParams(
          use_tc_tiling_on_sc=True, needs_layout_passes=False),
      scratch_types=[
          pltpu.VMEM((nsplit, G), jnp.int32),
          pltpu.VMEM((nsplit, G), jnp.int32),
          pltpu.VMEM((nsplit, G), jnp.int32),
          pltpu.VMEM((nsplit, G), jnp.int32),
          pltpu.VMEM((2, G, PAIR), jnp.float32),
          pltpu.VMEM((2, G, PAIR), jnp.float32),
          pltpu.VMEM((nsplit, G), jnp.float32),
          pltpu.VMEM((nsplit, G), jnp.float32),
          pltpu.VMEM((L,), jnp.float32),
          pltpu.SemaphoreType.DMA,
          pltpu.SemaphoreType.DMA,
          pltpu.SemaphoreType.DMA,
      ],
  )
  def k1(uidx_hbm, midx_hbm, pu_hbm, pm_hbm, ub_hbm, mb_hbm,
         part_hbm, bsum_hbm,
         uidx_v, midx_v, urow_v, mrow_v, urows_v, mrows_v, ubv, mbv, accv,
         sem_u, sem_m, sem_b):
    wid = lax.axis_index("s") * NC + lax.axis_index("c")
    row0 = wid * nsplit
    pltpu.sync_copy(uidx_hbm.at[pl.ds(row0, nsplit)], uidx_v)
    pltpu.sync_copy(midx_hbm.at[pl.ds(row0, nsplit)], midx_v)

    for j in range(nsplit):
      for s in range(G // L):
        sl = pl.ds(s * L, L)
        u = uidx_v[j, sl]
        urow_v[j, sl] = u - jnp.where(u >= SPLIT, SPLIT, 0)
        m = midx_v[j, sl]
        mrow_v[j, sl] = m - jnp.where(m >= SPLIT, SPLIT, 0)

    bias_copies = []
    for j in range(nsplit):
      bias_copies.append(pltpu.async_copy(
          ub_hbm.at[uidx_v.at[j]], ubv.at[j], sem_b))
      bias_copies.append(pltpu.async_copy(
          mb_hbm.at[midx_v.at[j]], mbv.at[j], sem_b))

    def gather(j, buf):
      cu = pltpu.async_copy(pu_hbm.at[urow_v.at[j]], urows_v.at[buf], sem_u)
      cm = pltpu.async_copy(pm_hbm.at[mrow_v.at[j]], mrows_v.at[buf], sem_m)
      return cu, cm

    pend = gather(0, 0)
    zero = jnp.zeros((L,), jnp.float32)
    acc = (zero, zero)
    lane = lax.iota(jnp.int32, L)
    for j in range(nsplit):
      buf = j % 2
      pend[0].wait()
      pend[1].wait()
      if j + 1 < nsplit:
        pend = gather(j + 1, (j + 1) % 2)
      a0, a1 = acc
      for g in range(G // L):
        sl = pl.ds(g * L, L)
        ridx = lane + (g * L)
        ucol0 = jnp.where(uidx_v[j, sl] >= SPLIT, EMBED, 0)
        mcol0 = jnp.where(midx_v[j, sl] >= SPLIT, EMBED, 0)

        def kbody(k, carry):
          b0, b1, uc, mc = carry
          u0 = plsc.load_gather(urows_v.at[buf], [ridx, uc])
          m0 = plsc.load_gather(mrows_v.at[buf], [ridx, mc])
          u1 = plsc.load_gather(urows_v.at[buf], [ridx, uc + 1])
          m1 = plsc.load_gather(mrows_v.at[buf], [ridx, mc + 1])
          return (b0 + u0 * m0, b1 + u1 * m1, uc + 2, mc + 2)

        a0, a1, _, _ = lax.fori_loop(
            0, EMBED // 2, kbody, (a0, a1, ucol0, mcol0), unroll=8)
      acc = (a0, a1)

    for c in bias_copies:
      c.wait()
    for j in range(nsplit):
      for s in range(G // L):
        sl = pl.ds(s * L, L)
        ubv[j, sl] = ubv[j, sl] + mbv[j, sl]
    pltpu.sync_copy(ubv, bsum_hbm.at[pl.ds(row0, nsplit)])

    accv[...] = acc[0] + acc[1]
    pltpu.sync_copy(accv, part_hbm.at[wid])

  return k1(uidx2d, midx2d, pu, pm, ub_flat, mb_flat)


def _tc_finish(part_ref, bsum_ref, out_ref):
  s = jnp.sum(part_ref[...])
  out_ref[...] = jax.nn.sigmoid(bsum_ref[...] + s)


def kernel(inputs, user_embedding, user_bias, movie_embedding, movie_bias):
  batch = inputs.shape[0]
  uidx2d = inputs[:, 0].reshape(batch // G, G)
  midx2d = inputs[:, 1].reshape(batch // G, G)
  ub_flat = user_bias.reshape(-1)
  mb_flat = movie_bias.reshape(-1)

  pu, pm = _tc_pack(user_embedding.T, movie_embedding.T)

  partials, bsum = _sc_gather_dot(
      uidx2d, midx2d, pu, pm, ub_flat, mb_flat, batch)

  out = pl.pallas_call(
      _tc_finish,
      out_shape=jax.ShapeDtypeStruct(bsum.shape, jnp.float32),
  )(partials, bsum)
  return out.reshape(batch, 1)


# speedup vs baseline: 1.5491x; 1.0343x over previous
import functools

import jax
import jax.numpy as jnp
from jax import lax
from jax.experimental import pallas as pl
from jax.experimental.pallas import tpu as pltpu
from jax.experimental.pallas import tpu_sc as plsc

NC = 2
NS = 16
L = 16
NW = NC * NS

G = 128
EMBED = 64
PAIR = 2 * EMBED
XCOLS = 2944
NBLK = 17
SPLIT = NBLK * XCOLS


def _pack_body(ua_ref, ub_ref, ma_ref, mb_ref, pu_ref, pm_ref):
  pu_ref[...] = jnp.concatenate(
      [jnp.transpose(ua_ref[...]), jnp.transpose(ub_ref[...])], axis=1)
  pm_ref[...] = jnp.concatenate(
      [jnp.transpose(ma_ref[...]), jnp.transpose(mb_ref[...])], axis=1)


def _tc_pack(ue_t, me_t):
  lo = pl.BlockSpec((EMBED, XCOLS), lambda b: (0, b))
  hi = pl.BlockSpec((EMBED, XCOLS), lambda b: (0, b + NBLK))
  out = pl.BlockSpec((XCOLS, PAIR), lambda b: (b, 0))
  return pl.pallas_call(
      _pack_body,
      grid=(NBLK,),
      in_specs=[lo, hi, lo, hi],
      out_specs=[out, out],
      out_shape=[jax.ShapeDtypeStruct((SPLIT, PAIR), jnp.float32)] * 2,
  )(ue_t, ue_t, me_t, me_t)


def _sc_gather_dot(uidx2d, midx2d, pu, pm, ub_flat, mb_flat, batch):
  chunk = batch // NW
  nsplit = chunk // G
  mesh = plsc.VectorSubcoreMesh(core_axis_name="c", subcore_axis_name="s")

  @functools.partial(
      pl.kernel,
      out_type=(
          jax.ShapeDtypeStruct((NW, L), jnp.float32),
          jax.ShapeDtypeStruct((batch // G, G), jnp.float32)
      ),
      mesh=mesh,
      compiler_params=pltpu.CompilerParams(
          use_tc_tiling_on_sc=True, needs_layout_passes=False),
      scratch_types=[
          pltpu.VMEM((nsplit, G), jnp.int32),
          pltpu.VMEM((nsplit, G), jnp.int32),
          pltpu.VMEM((nsplit, G), jnp.int32),
          pltpu.VMEM((nsplit, G), jnp.int32),
          pltpu.VMEM((chunk, PAIR), jnp.float32),
          pltpu.VMEM((2, G, PAIR), jnp.float32),
          pltpu.VMEM((nsplit, G), jnp.float32),
          pltpu.VMEM((nsplit, G), jnp.float32),
          pltpu.VMEM((L,), jnp.float32),
          pltpu.SemaphoreType.DMA,
          pltpu.SemaphoreType.DMA,
          pltpu.SemaphoreType.DMA,
      ],
  )
  def k1(uidx_hbm, midx_hbm, pu_hbm, pm_hbm, ub_hbm, mb_hbm,
         part_hbm, bsum_hbm,
         uidx_v, midx_v, urow_v, mrow_v, urows_v, mrows_v, ubv, mbv, accv,
         sem_u, sem_m, sem_b):
    wid = lax.axis_index("s") * NC + lax.axis_index("c")
    row0 = wid * nsplit
    pltpu.sync_copy(uidx_hbm.at[pl.ds(row0, nsplit)], uidx_v)
    pltpu.sync_copy(midx_hbm.at[pl.ds(row0, nsplit)], midx_v)

    for j in range(nsplit):
      for s in range(G // L):
        sl = pl.ds(s * L, L)
        u = uidx_v[j, sl]
        urow_v[j, sl] = u - jnp.where(u >= SPLIT, SPLIT, 0)
        m = midx_v[j, sl]
        mrow_v[j, sl] = m - jnp.where(m >= SPLIT, SPLIT, 0)

    u_copies = [
        pltpu.async_copy(pu_hbm.at[urow_v.at[j]],
                         urows_v.at[pl.ds(j * G, G)], sem_u)
        for j in range(nsplit)
    ]

    def gather_m(j, buf):
      return pltpu.async_copy(pm_hbm.at[mrow_v.at[j]], mrows_v.at[buf], sem_m)

    pend = gather_m(0, 0)
    bias_copies = []
    for j in range(nsplit):
      bias_copies.append(pltpu.async_copy(
          ub_hbm.at[uidx_v.at[j]], ubv.at[j], sem_b))
      bias_copies.append(pltpu.async_copy(
          mb_hbm.at[midx_v.at[j]], mbv.at[j], sem_b))

    zero = jnp.zeros((L,), jnp.float32)
    acc = (zero, zero)
    lane = lax.iota(jnp.int32, L)
    for j in range(nsplit):
      buf = j % 2
      u_copies[j].wait()
      pend.wait()
      if j + 1 < nsplit:
        pend = gather_m(j + 1, (j + 1) % 2)
      a0, a1 = acc
      for g in range(G // L):
        sl = pl.ds(g * L, L)
        ridx = lane + (j * G + g * L)
        mridx = lane + (g * L)
        ucol0 = jnp.where(uidx_v[j, sl] >= SPLIT, EMBED, 0)
        mcol0 = jnp.where(midx_v[j, sl] >= SPLIT, EMBED, 0)

        def kbody(k, carry):
          b0, b1, uc, mc = carry
          u0 = plsc.load_gather(urows_v, [ridx, uc])
          m0 = plsc.load_gather(mrows_v.at[buf], [mridx, mc])
          u1 = plsc.load_gather(urows_v, [ridx, uc + 1])
          m1 = plsc.load_gather(mrows_v.at[buf], [mridx, mc + 1])
          return (b0 + u0 * m0, b1 + u1 * m1, uc + 2, mc + 2)

        a0, a1, _, _ = lax.fori_loop(
            0, EMBED // 2, kbody, (a0, a1, ucol0, mcol0), unroll=8)
      acc = (a0, a1)

    for c in bias_copies:
      c.wait()
    for j in range(nsplit):
      for s in range(G // L):
        sl = pl.ds(s * L, L)
        ubv[j, sl] = ubv[j, sl] + mbv[j, sl]
    pltpu.sync_copy(ubv, bsum_hbm.at[pl.ds(row0, nsplit)])

    accv[...] = acc[0] + acc[1]
    pltpu.sync_copy(accv, part_hbm.at[wid])

  return k1(uidx2d, midx2d, pu, pm, ub_flat, mb_flat)


def _tc_finish(part_ref, bsum_ref, out_ref):
  s = jnp.sum(part_ref[...])
  out_ref[...] = jax.nn.sigmoid(bsum_ref[...] + s)


def kernel(inputs, user_embedding, user_bias, movie_embedding, movie_bias):
  batch = inputs.shape[0]
  uidx2d = inputs[:, 0].reshape(batch // G, G)
  midx2d = inputs[:, 1].reshape(batch // G, G)
  ub_flat = user_bias.reshape(-1)
  mb_flat = movie_bias.reshape(-1)

  pu, pm = _tc_pack(user_embedding.T, movie_embedding.T)

  partials, bsum = _sc_gather_dot(
      uidx2d, midx2d, pu, pm, ub_flat, mb_flat, batch)

  out = pl.pallas_call(
      _tc_finish,
      out_shape=jax.ShapeDtypeStruct(bsum.shape, jnp.float32),
  )(partials, bsum)
  return out.reshape(batch, 1)


# speedup vs baseline: 2.0824x; 1.3443x over previous
import functools

import jax
import jax.numpy as jnp
from jax import lax
from jax.experimental import pallas as pl
from jax.experimental.pallas import tpu as pltpu
from jax.experimental.pallas import tpu_sc as plsc

NC = 2
NS = 16
L = 16
NW = NC * NS

G = 128
EMBED = 64
PAIR = 2 * EMBED
XCOLS = 2944
NBLK = 17
SPLIT = NBLK * XCOLS


def _pack_body(ua_ref, ub_ref, ma_ref, mb_ref, pu_ref, pm_ref):
  pu_ref[...] = jnp.concatenate(
      [jnp.transpose(ua_ref[...]), jnp.transpose(ub_ref[...])], axis=1)
  pm_ref[...] = jnp.concatenate(
      [jnp.transpose(ma_ref[...]), jnp.transpose(mb_ref[...])], axis=1)


def _tc_pack(ue_t, me_t):
  lo = pl.BlockSpec((EMBED, XCOLS), lambda b: (0, b))
  hi = pl.BlockSpec((EMBED, XCOLS), lambda b: (0, b + NBLK))
  out = pl.BlockSpec((XCOLS, PAIR), lambda b: (b, 0))
  return pl.pallas_call(
      _pack_body,
      grid=(NBLK,),
      in_specs=[lo, hi, lo, hi],
      out_specs=[out, out],
      out_shape=[jax.ShapeDtypeStruct((SPLIT, PAIR), jnp.float32)] * 2,
  )(ue_t, ue_t, me_t, me_t)


def _sc_gather_dot(uidx2d, midx2d, pu64, pm64, ub_flat, mb_flat, batch):
  chunk = batch // NW
  nsplit = chunk // G
  mesh = plsc.VectorSubcoreMesh(core_axis_name="c", subcore_axis_name="s")

  @functools.partial(
      pl.kernel,
      out_type=(
          jax.ShapeDtypeStruct((NW, L), jnp.float32),
          jax.ShapeDtypeStruct((batch // G, G), jnp.float32)
      ),
      mesh=mesh,
      compiler_params=pltpu.CompilerParams(use_tc_tiling_on_sc=False),
      scratch_types=[
          pltpu.VMEM((nsplit, G), jnp.int32),
          pltpu.VMEM((nsplit, G), jnp.int32),
          pltpu.VMEM((nsplit, G), jnp.int32),
          pltpu.VMEM((nsplit, G), jnp.int32),
          pltpu.VMEM((chunk, EMBED), jnp.float32),
          pltpu.VMEM((2, G, EMBED), jnp.float32),
          pltpu.VMEM((nsplit, G), jnp.float32),
          pltpu.VMEM((nsplit, G), jnp.float32),
          pltpu.VMEM((L,), jnp.float32),
          pltpu.SemaphoreType.DMA,
          pltpu.SemaphoreType.DMA,
          pltpu.SemaphoreType.DMA,
      ],
  )
  def k1(uidx_hbm, midx_hbm, pu_hbm, pm_hbm, ub_hbm, mb_hbm,
         part_hbm, bsum_hbm,
         uidx_v, midx_v, urow_v, mrow_v, urows_v, mrows_v, ubv, mbv, accv,
         sem_u, sem_m, sem_b):
    wid = lax.axis_index("s") * NC + lax.axis_index("c")
    row0 = wid * nsplit
    pltpu.sync_copy(uidx_hbm.at[pl.ds(row0, nsplit)], uidx_v)
    pltpu.sync_copy(midx_hbm.at[pl.ds(row0, nsplit)], midx_v)

    for j in range(nsplit):
      for s in range(G // L):
        sl = pl.ds(s * L, L)
        u = uidx_v[j, sl]
        urow_v[j, sl] = 2 * u - jnp.where(u >= SPLIT, 2 * SPLIT - 1, 0)
        m = midx_v[j, sl]
        mrow_v[j, sl] = 2 * m - jnp.where(m >= SPLIT, 2 * SPLIT - 1, 0)

    u_copies = [
        pltpu.async_copy(pu_hbm.at[urow_v.at[j]],
                         urows_v.at[pl.ds(j * G, G)], sem_u)
        for j in range(nsplit)
    ]

    def gather_m(j, buf):
      return pltpu.async_copy(pm_hbm.at[mrow_v.at[j]], mrows_v.at[buf], sem_m)

    pend = gather_m(0, 0)
    bias_copies = []
    for j in range(nsplit):
      bias_copies.append(pltpu.async_copy(
          ub_hbm.at[uidx_v.at[j]], ubv.at[j], sem_b))
      bias_copies.append(pltpu.async_copy(
          mb_hbm.at[midx_v.at[j]], mbv.at[j], sem_b))

    zero = jnp.zeros((L,), jnp.float32)
    acc = (zero, zero, zero, zero)
    for j in range(nsplit):
      buf = j % 2
      u_copies[j].wait()
      pend.wait()
      if j + 1 < nsplit:
        pend = gather_m(j + 1, (j + 1) % 2)

      def dot_body(i, accs):
        a0, a1, a2, a3 = accs
        a0 = a0 + (urows_v[j * G + i, pl.ds(0, L)]
                   * mrows_v[buf, i, pl.ds(0, L)])
        a1 = a1 + (urows_v[j * G + i, pl.ds(L, L)]
                   * mrows_v[buf, i, pl.ds(L, L)])
        a2 = a2 + (urows_v[j * G + i, pl.ds(2 * L, L)]
                   * mrows_v[buf, i, pl.ds(2 * L, L)])
        a3 = a3 + (urows_v[j * G + i, pl.ds(3 * L, L)]
                   * mrows_v[buf, i, pl.ds(3 * L, L)])
        return (a0, a1, a2, a3)

      acc = lax.fori_loop(0, G, dot_body, acc, unroll=4)

    for c in bias_copies:
      c.wait()
    for j in range(nsplit):
      for s in range(G // L):
        sl = pl.ds(s * L, L)
        ubv[j, sl] = ubv[j, sl] + mbv[j, sl]
    pltpu.sync_copy(ubv, bsum_hbm.at[pl.ds(row0, nsplit)])

    accv[...] = (acc[0] + acc[1]) + (acc[2] + acc[3])
    pltpu.sync_copy(accv, part_hbm.at[wid])

  return k1(uidx2d, midx2d, pu64, pm64, ub_flat, mb_flat)


def _tc_finish(part_ref, bsum_ref, out_ref):
  s = jnp.sum(part_ref[...])
  out_ref[...] = jax.nn.sigmoid(bsum_ref[...] + s)


def kernel(inputs, user_embedding, user_bias, movie_embedding, movie_bias):
  batch = inputs.shape[0]
  uidx2d = inputs[:, 0].reshape(batch // G, G)
  midx2d = inputs[:, 1].reshape(batch // G, G)
  ub_flat = user_bias.reshape(-1)
  mb_flat = movie_bias.reshape(-1)

  pu, pm = _tc_pack(user_embedding.T, movie_embedding.T)
  pu64 = pu.reshape(2 * SPLIT, EMBED)
  pm64 = pm.reshape(2 * SPLIT, EMBED)

  partials, bsum = _sc_gather_dot(
      uidx2d, midx2d, pu64, pm64, ub_flat, mb_flat, batch)

  out = pl.pallas_call(
      _tc_finish,
      out_shape=jax.ShapeDtypeStruct(bsum.shape, jnp.float32),
  )(partials, bsum)
  return out.reshape(batch, 1)


# speedup vs baseline: 2.2771x; 1.0935x over previous
import functools

import jax
import jax.numpy as jnp
from jax import lax
from jax.experimental import pallas as pl
from jax.experimental.pallas import tpu as pltpu
from jax.experimental.pallas import tpu_sc as plsc

NC = 2
NS = 16
L = 16
NW = NC * NS

G = 128
EMBED = 64
PAIR = 2 * EMBED
XCOLS = 2944
NBLK = 17
SPLIT = NBLK * XCOLS


def _pack_body(ua_ref, ub_ref, ma_ref, mb_ref, pu_ref, pm_ref):
  r = lax.broadcasted_iota(jnp.int32, (PAIR, PAIR), 0)
  c = lax.broadcasted_iota(jnp.int32, (PAIR, PAIR), 1)
  eye = (r == c).astype(jnp.bfloat16)
  dn = (((0,), (0,)), ((), ()))

  def t(x):
    h = x.astype(jnp.bfloat16)
    r1 = x - h.astype(jnp.float32)
    m = r1.astype(jnp.bfloat16)
    lo = (r1 - m.astype(jnp.float32)).astype(jnp.bfloat16)
    ph = lax.dot_general(h, eye, dn, preferred_element_type=jnp.float32)
    pm = lax.dot_general(m, eye, dn, preferred_element_type=jnp.float32)
    pl_ = lax.dot_general(lo, eye, dn, preferred_element_type=jnp.float32)
    return (ph + pm) + pl_

  pu_ref[...] = t(jnp.concatenate([ua_ref[...], ub_ref[...]], axis=0))
  pm_ref[...] = t(jnp.concatenate([ma_ref[...], mb_ref[...]], axis=0))


def _tc_pack(ue_t, me_t):
  lo = pl.BlockSpec((EMBED, XCOLS), lambda b: (0, b))
  hi = pl.BlockSpec((EMBED, XCOLS), lambda b: (0, b + NBLK))
  out = pl.BlockSpec((XCOLS, PAIR), lambda b: (b, 0))
  return pl.pallas_call(
      _pack_body,
      grid=(NBLK,),
      in_specs=[lo, hi, lo, hi],
      out_specs=[out, out],
      out_shape=[jax.ShapeDtypeStruct((SPLIT, PAIR), jnp.float32)] * 2,
  )(ue_t, ue_t, me_t, me_t)


def _sc_gather_dot(idxv, pu64, pm64, ub_flat, mb_flat, batch):
  chunk = batch // NW
  nsplit = chunk // G
  mesh = plsc.VectorSubcoreMesh(core_axis_name="c", subcore_axis_name="s")

  @functools.partial(
      pl.kernel,
      out_type=(
          jax.ShapeDtypeStruct((NW, L), jnp.float32),
          jax.ShapeDtypeStruct((batch // G, G), jnp.float32)
      ),
      mesh=mesh,
      compiler_params=pltpu.CompilerParams(use_tc_tiling_on_sc=False),
      scratch_types=[
          pltpu.VMEM((nsplit, G), jnp.int32),
          pltpu.VMEM((nsplit, G), jnp.int32),
          pltpu.VMEM((nsplit, G), jnp.int32),
          pltpu.VMEM((nsplit, G), jnp.int32),
          pltpu.VMEM((chunk, EMBED), jnp.float32),
          pltpu.VMEM((2, G, EMBED), jnp.float32),
          pltpu.VMEM((nsplit, G), jnp.float32),
          pltpu.VMEM((nsplit, G), jnp.float32),
          pltpu.VMEM((L,), jnp.float32),
          pltpu.SemaphoreType.DMA,
          pltpu.SemaphoreType.DMA,
          pltpu.SemaphoreType.DMA,
      ],
  )
  def k1(idx_hbm, pu_hbm, pm_hbm, ub_hbm, mb_hbm,
         part_hbm, bsum_hbm,
         uidx_v, midx_v, urow_v, mrow_v, urows_v, mrows_v, ubv, mbv, accv,
         sem_u, sem_m, sem_b):
    wid = lax.axis_index("s") * NC + lax.axis_index("c")
    row0 = wid * nsplit
    pltpu.sync_copy(idx_hbm.at[pl.ds(row0, nsplit), 0], uidx_v)
    pltpu.sync_copy(idx_hbm.at[pl.ds(row0, nsplit), 1], midx_v)

    for j in range(nsplit):
      for s in range(G // L):
        sl = pl.ds(s * L, L)
        u = uidx_v[j, sl]
        urow_v[j, sl] = 2 * u - jnp.where(u >= SPLIT, 2 * SPLIT - 1, 0)
        m = midx_v[j, sl]
        mrow_v[j, sl] = 2 * m - jnp.where(m >= SPLIT, 2 * SPLIT - 1, 0)

    u_copies = [
        pltpu.async_copy(pu_hbm.at[urow_v.at[j]],
                         urows_v.at[pl.ds(j * G, G)], sem_u)
        for j in range(nsplit)
    ]

    def gather_m(j, buf):
      return pltpu.async_copy(pm_hbm.at[mrow_v.at[j]], mrows_v.at[buf], sem_m)

    pend = gather_m(0, 0)
    bias_copies = []
    for j in range(nsplit):
      bias_copies.append(pltpu.async_copy(
          ub_hbm.at[uidx_v.at[j]], ubv.at[j], sem_b))
      bias_copies.append(pltpu.async_copy(
          mb_hbm.at[midx_v.at[j]], mbv.at[j], sem_b))

    zero = jnp.zeros((L,), jnp.float32)
    acc = (zero, zero, zero, zero)
    for j in range(nsplit):
      buf = j % 2
      u_copies[j].wait()
      pend.wait()
      if j + 1 < nsplit:
        pend = gather_m(j + 1, (j + 1) % 2)

      def dot_body(i, accs):
        a0, a1, a2, a3 = accs
        a0 = a0 + (urows_v[j * G + i, pl.ds(0, L)]
                   * mrows_v[buf, i, pl.ds(0, L)])
        a1 = a1 + (urows_v[j * G + i, pl.ds(L, L)]
                   * mrows_v[buf, i, pl.ds(L, L)])
        a2 = a2 + (urows_v[j * G + i, pl.ds(2 * L, L)]
                   * mrows_v[buf, i, pl.ds(2 * L, L)])
        a3 = a3 + (urows_v[j * G + i, pl.ds(3 * L, L)]
                   * mrows_v[buf, i, pl.ds(3 * L, L)])
        return (a0, a1, a2, a3)

      acc = lax.fori_loop(0, G, dot_body, acc, unroll=4)

    for c in bias_copies:
      c.wait()
    for j in range(nsplit):
      for s in range(G // L):
        sl = pl.ds(s * L, L)
        ubv[j, sl] = ubv[j, sl] + mbv[j, sl]
    pltpu.sync_copy(ubv, bsum_hbm.at[pl.ds(row0, nsplit)])

    accv[...] = (acc[0] + acc[1]) + (acc[2] + acc[3])
    pltpu.sync_copy(accv, part_hbm.at[wid])

  return k1(idxv, pu64, pm64, ub_flat, mb_flat)


def _tc_finish(part_ref, bsum_ref, out_ref):
  s = jnp.sum(part_ref[...])
  out_ref[...] = jax.nn.sigmoid(bsum_ref[...] + s)


def kernel(inputs, user_embedding, user_bias, movie_embedding, movie_bias):
  batch = inputs.shape[0]
  idxv = inputs.reshape(batch // G, G, 2).transpose(0, 2, 1)
  ub_flat = user_bias.reshape(-1)
  mb_flat = movie_bias.reshape(-1)

  pu, pm = _tc_pack(user_embedding.T, movie_embedding.T)
  pu64 = pu.reshape(2 * SPLIT, EMBED)
  pm64 = pm.reshape(2 * SPLIT, EMBED)

  partials, bsum = _sc_gather_dot(
      idxv, pu64, pm64, ub_flat, mb_flat, batch)

  out = pl.pallas_call(
      _tc_finish,
      out_shape=jax.ShapeDtypeStruct(bsum.shape, jnp.float32),
  )(partials, bsum)
  return out.reshape(batch, 1)
